# baseline (speedup 1.0000x reference)
import functools

import jax
import jax.numpy as jnp
from jax import lax
from jax.experimental import pallas as pl
from jax.experimental.pallas import tpu as pltpu
from jax.experimental.pallas import tpu_sc as plsc

B, S, D = 4096, 50, 128
SP = 56
NC, NS = 2, 16
NW = NC * NS
BPW = B // NW
NSTEP = BPW
NBUF = 8
NGRP = NSTEP // NBUF


def _emb_body(x_hbm, w_hbm, out_hbm, idx_v, rows_v, gsems, osems):
    wid = lax.axis_index("s") * NC + lax.axis_index("c")
    b0 = wid * BPW

    pltpu.sync_copy(x_hbm.at[wid], idx_v)

    def gstart(b, step):
        pltpu.async_copy(w_hbm.at[idx_v.at[step]],
                         rows_v.at[b, pl.ds(0, S)], gsems.at[b])

    def gwait(b):
        pltpu.make_async_copy(w_hbm.at[idx_v.at[0]],
                              rows_v.at[b, pl.ds(0, S)], gsems.at[b]).wait()

    def wstart(b, step):
        pltpu.async_copy(rows_v.at[b], out_hbm.at[b0 + step], osems.at[b])

    def wwait(b):
        pltpu.make_async_copy(rows_v.at[b], out_hbm.at[b0],
                              osems.at[b]).wait()

    for b in range(NBUF):
        gstart(b, b)

    def group(g, _):
        for b in range(NBUF):
            gwait(b)
            wstart(b, g * NBUF + b)
        for b in range(NBUF):
            wwait(b)
            nxt = (g + 1) * NBUF + b

            @pl.when(g < NGRP - 1)
            def _():
                gstart(b, nxt)
        return _

    lax.fori_loop(0, NGRP, group, None)


@jax.jit
def _emb(xw, weight):
    kern = pl.kernel(
        _emb_body,
        out_type=jax.ShapeDtypeStruct((B, SP, D), jnp.float32),
        mesh=plsc.VectorSubcoreMesh(core_axis_name="c", subcore_axis_name="s"),
        scratch_types=[
            pltpu.VMEM((NSTEP, S), jnp.int32),
            pltpu.VMEM((NBUF, SP, D), jnp.float32),
            pltpu.SemaphoreType.DMA((NBUF,)),
            pltpu.SemaphoreType.DMA((NBUF,)),
        ],
    )
    return kern(xw, weight)


def kernel(x, weight):
    xw = x.astype(jnp.int32).reshape(NW, NSTEP, S)
    return _emb(xw, weight)[:, :S, :]


# speedup vs baseline: 1.1710x; 1.1710x over previous
import functools

import jax
import jax.numpy as jnp
from jax import lax
from jax.experimental import pallas as pl
from jax.experimental.pallas import tpu as pltpu
from jax.experimental.pallas import tpu_sc as plsc

B, S, D = 4096, 50, 128
SP = 56
NC, NS = 2, 16
NW = NC * NS
BPW = B // NW
NSTEP = BPW
NBUF = 8
NGRP = NSTEP // NBUF


def _emb_body(x_hbm, w_hbm, out_hbm, idx_v, rows_v, gsems, osems):
    wid = lax.axis_index("s") * NC + lax.axis_index("c")
    b0 = wid * BPW

    pltpu.sync_copy(x_hbm.at[wid], idx_v)

    def gstart(b, step):
        pltpu.async_copy(w_hbm.at[idx_v.at[step]], rows_v.at[b], gsems.at[b])

    def gwait(b):
        pltpu.make_async_copy(w_hbm.at[idx_v.at[0]], rows_v.at[b],
                              gsems.at[b]).wait()

    def wstart(b, step):
        pltpu.async_copy(rows_v.at[b], out_hbm.at[b0 + step], osems.at[b])

    def wwait(b):
        pltpu.make_async_copy(rows_v.at[b], out_hbm.at[b0],
                              osems.at[b]).wait()

    for b in range(NBUF):
        gstart(b, b)

    def group(g, _):
        for b in range(NBUF):
            gwait(b)
            wstart(b, g * NBUF + b)
        for b in range(NBUF):
            wwait(b)
            nxt = (g + 1) * NBUF + b

            @pl.when(g < NGRP - 1)
            def _():
                gstart(b, nxt)
        return _

    lax.fori_loop(0, NGRP, group, None)


@jax.jit
def _emb(xw, weight):
    kern = pl.kernel(
        _emb_body,
        out_type=jax.ShapeDtypeStruct((B, S, D), jnp.float32),
        mesh=plsc.VectorSubcoreMesh(core_axis_name="c", subcore_axis_name="s"),
        compiler_params=pltpu.CompilerParams(use_tc_tiling_on_sc=True),
        scratch_types=[
            pltpu.VMEM((NSTEP, S), jnp.int32),
            pltpu.VMEM((NBUF, S, D), jnp.float32),
            pltpu.SemaphoreType.DMA((NBUF,)),
            pltpu.SemaphoreType.DMA((NBUF,)),
        ],
    )
    return kern(xw, weight)


def kernel(x, weight):
    xw = x.astype(jnp.int32).reshape(NW, NSTEP, S)
    return _emb(xw, weight)
